# baseline (speedup 1.0000x reference)
import functools

import jax
import jax.numpy as jnp
from jax import lax
from jax.experimental import pallas as pl
from jax.experimental.pallas import tpu as pltpu
from jax.experimental.pallas import tpu_sc as plsc

NUM_OPERATIONS = 1000000
EMBED_DIM = 64
BATCH = 16384

NC = 2
NS = 16
NW = NC * NS
B_PER_W = BATCH // NW
CHUNK = 128
NCHUNK = B_PER_W // CHUNK
LANES = 16
NSLICE = B_PER_W // LANES


def _rsqrt_newton(x):
    i = lax.bitcast_convert_type(x, jnp.int32)
    i = jnp.int32(0x5F3759DF) - (i >> 1)
    y = lax.bitcast_convert_type(i, jnp.float32)
    for _ in range(3):
        y = y * (1.5 - 0.5 * x * y * y)
    return y


def _sc_body(tab_hbm, idx_hbm, out_hbm, idx_v, rows_v, cols_v, sem):
    wid = lax.axis_index("s") * NC + lax.axis_index("c")
    base = wid * B_PER_W

    pltpu.sync_copy(idx_hbm.at[pl.ds(base, B_PER_W)], idx_v)

    copies = [
        pltpu.async_copy(
            tab_hbm.at[idx_v.at[pl.ds(j * CHUNK, CHUNK)]],
            rows_v.at[pl.ds(j * CHUNK, CHUNK)],
            sem,
        )
        for j in range(NCHUNK)
    ]
    for c in copies:
        c.wait()

    lanes16 = lax.iota(jnp.int32, LANES)

    def norm_body(s, carry):
        rows16 = lanes16 + s * LANES
        col = pl.ds(s * LANES, LANES)
        acc = jnp.zeros((LANES,), jnp.float32)
        for c in range(EMBED_DIM):
            x = plsc.load_gather(
                rows_v, [rows16, jnp.full((LANES,), c, jnp.int32)]
            )
            acc = acc + x * x
            cols_v[c, col] = x
        acc = jnp.maximum(acc, jnp.float32(1e-30))
        inv = jnp.minimum(_rsqrt_newton(acc), jnp.float32(1e12))
        for c in range(EMBED_DIM):
            cols_v[c, col] = cols_v[c, col] * inv
        return carry

    lax.fori_loop(0, NSLICE, norm_body, 0)

    pltpu.sync_copy(cols_v, out_hbm.at[:, pl.ds(base, B_PER_W)])


@functools.lru_cache(maxsize=None)
def _build():
    mesh = plsc.VectorSubcoreMesh(
        core_axis_name="c", subcore_axis_name="s", num_cores=NC, num_subcores=NS
    )
    return pl.kernel(
        _sc_body,
        out_type=jax.ShapeDtypeStruct((EMBED_DIM, BATCH), jnp.float32),
        mesh=mesh,
        scratch_types=[
            pltpu.VMEM((B_PER_W,), jnp.int32),
            pltpu.VMEM((B_PER_W, EMBED_DIM), jnp.float32),
            pltpu.VMEM((EMBED_DIM, B_PER_W), jnp.float32),
            pltpu.SemaphoreType.DMA,
        ],
        compiler_params=pltpu.CompilerParams(use_tc_tiling_on_sc=False, needs_layout_passes=False, skip_device_barrier=True),
    )


RB = 8192


def _tc_transpose_body(x_ref, o_ref):
    eye = jnp.eye(EMBED_DIM, dtype=jnp.float32)
    o_ref[...] = lax.dot_general(
        x_ref[...],
        eye,
        dimension_numbers=(((0,), (0,)), ((), ())),
        preferred_element_type=jnp.float32,
    )


@functools.lru_cache(maxsize=None)
def _build_tc_transpose():
    return pl.pallas_call(
        _tc_transpose_body,
        grid=(pl.cdiv(NUM_OPERATIONS, RB),),
        in_specs=[pl.BlockSpec((EMBED_DIM, RB), lambda i: (0, i))],
        out_specs=pl.BlockSpec((RB, EMBED_DIM), lambda i: (i, 0)),
        out_shape=jax.ShapeDtypeStruct(
            (NUM_OPERATIONS, EMBED_DIM), jnp.float32
        ),
    )


def kernel(operation_ids, table):
    idx = operation_ids.astype(jnp.int32)
    lin = _build_tc_transpose()(table.T)
    out_t = _build()(lin, idx)
    return out_t.T


# speedup vs baseline: 1.0131x; 1.0131x over previous
import functools

import jax
import jax.numpy as jnp
from jax import lax
from jax.experimental import pallas as pl
from jax.experimental.pallas import tpu as pltpu
from jax.experimental.pallas import tpu_sc as plsc

NUM_OPERATIONS = 1000000
EMBED_DIM = 64
BATCH = 16384

NC = 2
NS = 16
NW = NC * NS
B_PER_W = BATCH // NW
CHUNK = 128
NCHUNK = B_PER_W // CHUNK
LANES = 16
NSLICE = B_PER_W // LANES


def _rsqrt_newton(x):
    i = lax.bitcast_convert_type(x, jnp.int32)
    i = jnp.int32(0x5F3759DF) - (i >> 1)
    y = lax.bitcast_convert_type(i, jnp.float32)
    for _ in range(3):
        y = y * (1.5 - 0.5 * x * y * y)
    return y


def _sc_body(tab_hbm, idx_hbm, out_hbm, idx_v, rows_v, cols_v, sem):
    wid = lax.axis_index("s") * NC + lax.axis_index("c")
    base = wid * B_PER_W

    pltpu.sync_copy(idx_hbm.at[pl.ds(base, B_PER_W)], idx_v)

    copies = [
        pltpu.async_copy(
            tab_hbm.at[idx_v.at[pl.ds(j * CHUNK, CHUNK)]],
            rows_v.at[pl.ds(j * CHUNK, CHUNK)],
            sem,
        )
        for j in range(NCHUNK)
    ]
    for c in copies:
        c.wait()

    lanes16 = lax.iota(jnp.int32, LANES)

    def norm_body(s, carry):
        rows16 = lanes16 + s * LANES
        col = pl.ds(s * LANES, LANES)
        acc = jnp.zeros((LANES,), jnp.float32)
        for c in range(EMBED_DIM):
            x = plsc.load_gather(
                rows_v, [rows16, jnp.full((LANES,), c, jnp.int32)]
            )
            acc = acc + x * x
            cols_v[c, col] = x
        acc = jnp.maximum(acc, jnp.float32(1e-30))
        inv = jnp.minimum(_rsqrt_newton(acc), jnp.float32(1e12))
        for c in range(EMBED_DIM):
            cols_v[c, col] = cols_v[c, col] * inv
        return carry

    lax.fori_loop(0, NSLICE, norm_body, 0)

    pltpu.sync_copy(cols_v, out_hbm.at[:, pl.ds(base, B_PER_W)])


@functools.lru_cache(maxsize=None)
def _build():
    mesh = plsc.VectorSubcoreMesh(
        core_axis_name="c", subcore_axis_name="s", num_cores=NC, num_subcores=NS
    )
    return pl.kernel(
        _sc_body,
        out_type=jax.ShapeDtypeStruct((EMBED_DIM, BATCH), jnp.float32),
        mesh=mesh,
        scratch_types=[
            pltpu.VMEM((B_PER_W,), jnp.int32),
            pltpu.VMEM((B_PER_W, EMBED_DIM), jnp.float32),
            pltpu.VMEM((EMBED_DIM, B_PER_W), jnp.float32),
            pltpu.SemaphoreType.DMA,
        ],
        compiler_params=pltpu.CompilerParams(use_tc_tiling_on_sc=False, needs_layout_passes=False, skip_device_barrier=True),
    )


RB = 8192


def _tc_transpose_body(x_ref, o_ref):
    o_ref[...] = x_ref[...].T


@functools.lru_cache(maxsize=None)
def _build_tc_transpose():
    return pl.pallas_call(
        _tc_transpose_body,
        grid=(pl.cdiv(NUM_OPERATIONS, RB),),
        in_specs=[pl.BlockSpec((EMBED_DIM, RB), lambda i: (0, i))],
        out_specs=pl.BlockSpec((RB, EMBED_DIM), lambda i: (i, 0)),
        out_shape=jax.ShapeDtypeStruct(
            (NUM_OPERATIONS, EMBED_DIM), jnp.float32
        ),
    )


def kernel(operation_ids, table):
    idx = operation_ids.astype(jnp.int32)
    lin = _build_tc_transpose()(table.T)
    out_t = _build()(lin, idx)
    return out_t.T


# speedup vs baseline: 1.0870x; 1.0729x over previous
import functools

import jax
import jax.numpy as jnp
from jax import lax
from jax.experimental import pallas as pl
from jax.experimental.pallas import tpu as pltpu
from jax.experimental.pallas import tpu_sc as plsc

NUM_OPERATIONS = 1000000
EMBED_DIM = 64
BATCH = 16384

NC = 2
NS = 16
NW = NC * NS
B_PER_W = BATCH // NW
CHUNK = 128
NCHUNK = B_PER_W // CHUNK
LANES = 16
NSLICE = B_PER_W // LANES


def _rsqrt_newton(x):
    i = lax.bitcast_convert_type(x, jnp.int32)
    i = jnp.int32(0x5F3759DF) - (i >> 1)
    y = lax.bitcast_convert_type(i, jnp.float32)
    for _ in range(3):
        y = y * (1.5 - 0.5 * x * y * y)
    return y


def _sc_body(tab_hbm, idx_hbm, out_hbm, idx_v, rows_v, cols_v, sem):
    wid = lax.axis_index("s") * NC + lax.axis_index("c")
    base = wid * B_PER_W

    pltpu.sync_copy(idx_hbm.at[pl.ds(base, B_PER_W)], idx_v)

    copies = [
        pltpu.async_copy(
            tab_hbm.at[idx_v.at[pl.ds(j * CHUNK, CHUNK)]],
            rows_v.at[pl.ds(j * CHUNK, CHUNK)],
            sem,
        )
        for j in range(NCHUNK)
    ]
    for c in copies:
        c.wait()

    lanes16 = lax.iota(jnp.int32, LANES)

    def norm_body(s, carry):
        rows16 = lanes16 + s * LANES
        col = pl.ds(s * LANES, LANES)
        acc = jnp.zeros((LANES,), jnp.float32)
        for c in range(EMBED_DIM):
            x = plsc.load_gather(
                rows_v, [rows16, jnp.full((LANES,), c, jnp.int32)]
            )
            acc = acc + x * x
            cols_v[c, col] = x
        acc = jnp.maximum(acc, jnp.float32(1e-30))
        inv = jnp.minimum(_rsqrt_newton(acc), jnp.float32(1e12))
        for c in range(EMBED_DIM):
            cols_v[c, col] = cols_v[c, col] * inv
        return carry

    lax.fori_loop(0, NSLICE, norm_body, 0)

    pltpu.sync_copy(cols_v, out_hbm.at[:, pl.ds(base, B_PER_W)])


@functools.lru_cache(maxsize=None)
def _build():
    mesh = plsc.VectorSubcoreMesh(
        core_axis_name="c", subcore_axis_name="s", num_cores=NC, num_subcores=NS
    )
    return pl.kernel(
        _sc_body,
        out_type=jax.ShapeDtypeStruct((EMBED_DIM, BATCH), jnp.float32),
        mesh=mesh,
        scratch_types=[
            pltpu.VMEM((B_PER_W,), jnp.int32),
            pltpu.VMEM((B_PER_W, EMBED_DIM), jnp.float32),
            pltpu.VMEM((EMBED_DIM, B_PER_W), jnp.float32),
            pltpu.SemaphoreType.DMA,
        ],
        compiler_params=pltpu.CompilerParams(
            use_tc_tiling_on_sc=False, needs_layout_passes=False
        ),
    )


def kernel(operation_ids, table):
    idx = operation_ids.astype(jnp.int32)
    out_t = _build()(table, idx)
    return out_t.T


# speedup vs baseline: 1.0929x; 1.0055x over previous
import functools

import jax
import jax.numpy as jnp
from jax import lax
from jax.experimental import pallas as pl
from jax.experimental.pallas import tpu as pltpu
from jax.experimental.pallas import tpu_sc as plsc

NUM_OPERATIONS = 1000000
EMBED_DIM = 64
BATCH = 16384

NC = 2
NS = 16
NW = NC * NS
B_PER_W = BATCH // NW
CHUNK = 128
NCHUNK = B_PER_W // CHUNK
LANES = 16
NSLICE = B_PER_W // LANES


def _rsqrt_newton(x):
    i = lax.bitcast_convert_type(x, jnp.int32)
    i = jnp.int32(0x5F3759DF) - (i >> 1)
    y = lax.bitcast_convert_type(i, jnp.float32)
    for _ in range(3):
        y = y * (1.5 - 0.5 * x * y * y)
    return y


def _sc_body(tab_hbm, idx_hbm, out_hbm, idx_v, idx2_v, rows_v, cols_v, sem):
    wid = lax.axis_index("s") * NC + lax.axis_index("c")
    base = wid * B_PER_W

    pltpu.sync_copy(idx_hbm.at[pl.ds(base, B_PER_W)], idx_v)

    def halve(s, carry):
        sl = pl.ds(s * LANES, LANES)
        idx2_v[sl] = idx_v[sl] >> 1
        return carry

    lax.fori_loop(0, NSLICE, halve, 0)

    copies = [
        pltpu.async_copy(
            tab_hbm.at[idx2_v.at[pl.ds(j * CHUNK, CHUNK)]],
            rows_v.at[pl.ds(j * CHUNK, CHUNK)],
            sem,
        )
        for j in range(NCHUNK)
    ]
    for c in copies:
        c.wait()

    lanes16 = lax.iota(jnp.int32, LANES)

    def norm_body(s, carry):
        rows16 = lanes16 + s * LANES
        col = pl.ds(s * LANES, LANES)
        half = (idx_v[col] & 1) * EMBED_DIM
        acc = jnp.zeros((LANES,), jnp.float32)
        for c in range(EMBED_DIM):
            x = plsc.load_gather(rows_v, [rows16, half + c])
            acc = acc + x * x
            cols_v[c, col] = x
        acc = jnp.maximum(acc, jnp.float32(1e-30))
        inv = jnp.minimum(_rsqrt_newton(acc), jnp.float32(1e12))
        for c in range(EMBED_DIM):
            cols_v[c, col] = cols_v[c, col] * inv
        return carry

    lax.fori_loop(0, NSLICE, norm_body, 0)

    pltpu.sync_copy(cols_v, out_hbm.at[:, pl.ds(base, B_PER_W)])


@functools.lru_cache(maxsize=None)
def _build():
    mesh = plsc.VectorSubcoreMesh(
        core_axis_name="c", subcore_axis_name="s", num_cores=NC, num_subcores=NS
    )
    return pl.kernel(
        _sc_body,
        out_type=jax.ShapeDtypeStruct((EMBED_DIM, BATCH), jnp.float32),
        mesh=mesh,
        scratch_types=[
            pltpu.VMEM((B_PER_W,), jnp.int32),
            pltpu.VMEM((B_PER_W,), jnp.int32),
            pltpu.VMEM((B_PER_W, 2 * EMBED_DIM), jnp.float32),
            pltpu.VMEM((EMBED_DIM, B_PER_W), jnp.float32),
            pltpu.SemaphoreType.DMA,
        ],
        compiler_params=pltpu.CompilerParams(
            use_tc_tiling_on_sc=True, needs_layout_passes=False
        ),
    )


def kernel(operation_ids, table):
    idx = operation_ids.astype(jnp.int32)
    out_t = _build()(table.reshape(NUM_OPERATIONS // 2, 2 * EMBED_DIM), idx)
    return out_t.T


# speedup vs baseline: 1.1096x; 1.0152x over previous
import functools

import jax
import jax.numpy as jnp
from jax import lax
from jax.experimental import pallas as pl
from jax.experimental.pallas import tpu as pltpu
from jax.experimental.pallas import tpu_sc as plsc

NUM_OPERATIONS = 1000000
EMBED_DIM = 64
BATCH = 16384

NC = 2
NS = 16
NW = NC * NS
B_PER_W = BATCH // NW
CHUNK = 128
NCHUNK = B_PER_W // CHUNK
LANES = 16
VPR = EMBED_DIM // LANES


def _rsqrt_newton(x):
    i = lax.bitcast_convert_type(x, jnp.int32)
    i = jnp.int32(0x5F3759DF) - (i >> 1)
    y = lax.bitcast_convert_type(i, jnp.float32)
    for _ in range(3):
        y = y * (1.5 - 0.5 * x * y * y)
    return y


def _lane_sum(v):
    lanes = lax.iota(jnp.int32, LANES)
    for s in (8, 4, 2, 1):
        perm = (lanes + s) % LANES
        v = v + v.at[perm].get(mode="promise_in_bounds")
    return v


def _sc_body(table_hbm, idx_hbm, out_hbm, idx_v, rows_v, sem):
    wid = lax.axis_index("s") * NC + lax.axis_index("c")
    base = wid * B_PER_W

    pltpu.sync_copy(idx_hbm.at[pl.ds(base, B_PER_W)], idx_v)

    copies = [
        pltpu.async_copy(
            table_hbm.at[idx_v.at[pl.ds(j * CHUNK, CHUNK)]],
            rows_v.at[pl.ds(j * CHUNK, CHUNK)],
            sem,
        )
        for j in range(NCHUNK)
    ]
    for c in copies:
        c.wait()

    def row_body(i, carry):
        vs = [rows_v[i, pl.ds(k * LANES, LANES)] for k in range(VPR)]
        sq = vs[0] * vs[0]
        for k in range(1, VPR):
            sq = sq + vs[k] * vs[k]
        tot = _lane_sum(sq)
        tot = jnp.maximum(tot, jnp.float32(1e-30))
        inv = jnp.minimum(_rsqrt_newton(tot), jnp.float32(1e12))
        for k in range(VPR):
            rows_v[i, pl.ds(k * LANES, LANES)] = vs[k] * inv
        return carry

    lax.fori_loop(0, B_PER_W, row_body, 0, unroll=4)

    pltpu.sync_copy(rows_v, out_hbm.at[pl.ds(base, B_PER_W)])


@functools.lru_cache(maxsize=None)
def _build():
    mesh = plsc.VectorSubcoreMesh(
        core_axis_name="c", subcore_axis_name="s", num_cores=NC, num_subcores=NS
    )
    return pl.kernel(
        _sc_body,
        out_type=jax.ShapeDtypeStruct((BATCH, EMBED_DIM), jnp.float32),
        mesh=mesh,
        scratch_types=[
            pltpu.VMEM((B_PER_W,), jnp.int32),
            pltpu.VMEM((B_PER_W, EMBED_DIM), jnp.float32),
            pltpu.SemaphoreType.DMA,
        ],
        compiler_params=pltpu.CompilerParams(use_tc_tiling_on_sc=False),
    )


def kernel(operation_ids, table):
    idx = operation_ids.astype(jnp.int32)
    return _build()(table, idx)
